# baseline (device time: 66330 ns/iter reference)
import jax
import jax.numpy as jnp
from jax import lax
from jax.experimental import pallas as pl
from jax.experimental.pallas import tpu as pltpu

N_DEV = 8
SQ = 1024
SKV = 1024
DH = 128
H_PER = 8
D_MODEL = 1024
SCALE = 0.08838834764831843
WINDOW = 128
QT = 128
KW = 384
CHUNK = SQ // N_DEV
N_STEP = N_DEV - 1


def _body(x_ref, wq_ref, k_ref, v_ref, wo_ref, out_ref,
          ctx_ref, p_ref, bc_src, rs_buf, bc_buf,
          sc_send, sc_recv, bc_send, bc_recv):
    my = lax.axis_index("i")

    barrier = pltpu.get_barrier_semaphore()
    for p in range(N_DEV):
        @pl.when(p != my)
        def _():
            pl.semaphore_signal(barrier, inc=1, device_id=(p,),
                                device_id_type=pl.DeviceIdType.MESH)
    pl.semaphore_wait(barrier, N_DEV - 1)

    for k in range(N_DEV):
        t = lax.rem(my + 1 + k, N_DEV)
        rows = pl.ds(t * QT, QT)
        q_t = jnp.dot(x_ref[rows, :], wq_ref[...],
                      preferred_element_type=jnp.float32)
        kstart = jnp.clip(t * QT - WINDOW, 0, SKV - KW)
        q0 = t * QT
        qi = q0 + lax.broadcasted_iota(jnp.int32, (QT, KW), 0)
        ki = kstart + lax.broadcasted_iota(jnp.int32, (QT, KW), 1)
        mask = jnp.abs(qi - ki) <= WINDOW
        for h in range(H_PER):
            k_b = k_ref[h, pl.ds(kstart, KW), :]
            v_b = v_ref[h, pl.ds(kstart, KW), :]
            scores = lax.dot_general(
                q_t[:, h * DH:(h + 1) * DH], k_b, (((1,), (1,)), ((), ())),
                preferred_element_type=jnp.float32) * SCALE
            scores = jnp.where(mask, scores, -1e9)
            m = jnp.max(scores, axis=1, keepdims=True)
            w = jnp.exp(scores - m)
            w = w / jnp.sum(w, axis=1, keepdims=True)
            ctx_ref[rows, h * DH:(h + 1) * DH] = jnp.dot(
                w, v_b, preferred_element_type=jnp.float32)
        p_ref[rows, :] = jnp.dot(ctx_ref[rows, :], wo_ref[...],
                                 preferred_element_type=jnp.float32
                                 ).astype(jnp.bfloat16)
        if k < N_DEV - 1:
            pltpu.make_async_remote_copy(
                src_ref=p_ref.at[rows, :],
                dst_ref=rs_buf.at[my],
                send_sem=sc_send.at[k],
                recv_sem=sc_recv.at[my],
                device_id=(t,),
                device_id_type=pl.DeviceIdType.MESH,
            ).start()

    for p in range(N_DEV):
        @pl.when(p != my)
        def _():
            pltpu.make_async_remote_copy(
                src_ref=rs_buf.at[p], dst_ref=rs_buf.at[p],
                send_sem=sc_send.at[0], recv_sem=sc_recv.at[p],
                device_id=(my,), device_id_type=pl.DeviceIdType.MESH,
            ).wait_recv()
    for k in range(N_DEV - 1):
        t = lax.rem(my + 1 + k, N_DEV)
        pltpu.make_async_remote_copy(
            src_ref=p_ref.at[pl.ds(t * QT, QT), :],
            dst_ref=rs_buf.at[my],
            send_sem=sc_send.at[k],
            recv_sem=sc_recv.at[my],
            device_id=(t,), device_id_type=pl.DeviceIdType.MESH,
        ).wait_send()

    own = p_ref[pl.ds(my * CHUNK, CHUNK), :]
    red = jnp.zeros((CHUNK, D_MODEL), jnp.float32)
    for j in range(N_DEV):
        red = red + jnp.where(my == j, own, rs_buf[j]).astype(jnp.float32)
    out_ref[pl.ds(my * CHUNK, CHUNK), :] = red
    bc_src[...] = red.astype(jnp.bfloat16)

    for q in range(N_DEV):
        @pl.when(q != my)
        def _():
            pltpu.make_async_remote_copy(
                src_ref=bc_src,
                dst_ref=bc_buf.at[my],
                send_sem=bc_send.at[q],
                recv_sem=bc_recv.at[my],
                device_id=(q,),
                device_id_type=pl.DeviceIdType.MESH,
            ).start()
    for p in range(N_DEV):
        @pl.when(p != my)
        def _():
            pltpu.make_async_remote_copy(
                src_ref=bc_src, dst_ref=bc_buf.at[p],
                send_sem=bc_send.at[p], recv_sem=bc_recv.at[p],
                device_id=(my,), device_id_type=pl.DeviceIdType.MESH,
            ).wait_recv()
            out_ref[pl.ds(p * CHUNK, CHUNK), :] = (
                bc_buf[p].astype(jnp.float32))
    for q in range(N_DEV):
        @pl.when(q != my)
        def _():
            pltpu.make_async_remote_copy(
                src_ref=bc_src, dst_ref=bc_buf.at[my],
                send_sem=bc_send.at[q], recv_sem=bc_recv.at[my],
                device_id=(q,), device_id_type=pl.DeviceIdType.MESH,
            ).wait_send()


def kernel(x, Wq, K_ext, V_ext, Wo):
    i = lax.axis_index("i")
    x2 = x[0]
    wq = lax.dynamic_slice(Wq, (0, i * D_MODEL), (D_MODEL, H_PER * DH))
    wo = lax.dynamic_slice(Wo, (i * H_PER * DH, 0), (H_PER * DH, D_MODEL))
    kh = jnp.swapaxes(K_ext[0], 0, 1)
    vh = jnp.swapaxes(V_ext[0], 0, 1)

    out = pl.pallas_call(
        _body,
        out_shape=jax.ShapeDtypeStruct((SQ, D_MODEL), jnp.float32),
        in_specs=[pl.BlockSpec(memory_space=pltpu.VMEM)] * 5,
        out_specs=pl.BlockSpec(memory_space=pltpu.VMEM),
        scratch_shapes=[
            pltpu.VMEM((SQ, H_PER * DH), jnp.float32),
            pltpu.VMEM((SQ, D_MODEL), jnp.bfloat16),
            pltpu.VMEM((CHUNK, D_MODEL), jnp.bfloat16),
            pltpu.VMEM((N_DEV, CHUNK, D_MODEL), jnp.bfloat16),
            pltpu.VMEM((N_DEV, CHUNK, D_MODEL), jnp.bfloat16),
            pltpu.SemaphoreType.DMA((N_DEV,)),
            pltpu.SemaphoreType.DMA((N_DEV,)),
            pltpu.SemaphoreType.DMA((N_DEV,)),
            pltpu.SemaphoreType.DMA((N_DEV,)),
        ],
        compiler_params=pltpu.CompilerParams(collective_id=0),
    )(x2, wq, kh, vh, wo)
    return out[None]


# device time: 57141 ns/iter; 1.1608x vs baseline; 1.1608x over previous
import jax
import jax.numpy as jnp
from jax import lax
from jax.experimental import pallas as pl
from jax.experimental.pallas import tpu as pltpu

N_DEV = 8
SQ = 1024
SKV = 1024
DH = 128
H_PER = 8
D_MODEL = 1024
SCALE = 0.08838834764831843
WINDOW = 128
QT = 256
KW = 512
CHUNK = SQ // N_DEV
N_TILE = SQ // QT


def _body(x_ref, wq_ref, k_ref, v_ref, wo_ref, out_ref,
          ctx_ref, p_ref, bc_src, rs_buf, bc_buf,
          sc_send, sc_recv, bc_send, bc_recv):
    my = lax.axis_index("i")

    barrier = pltpu.get_barrier_semaphore()
    for p in range(N_DEV):
        @pl.when(p != my)
        def _():
            pl.semaphore_signal(barrier, inc=1, device_id=(p,),
                                device_id_type=pl.DeviceIdType.MESH)
    pl.semaphore_wait(barrier, N_DEV - 1)

    my_tile = lax.div(my, 2)
    for k in range(N_TILE):
        t = lax.rem(my_tile + 1 + k, N_TILE)
        q0 = t * QT
        rows = pl.ds(q0, QT)
        q_t = jnp.dot(x_ref[rows, :], wq_ref[...],
                      preferred_element_type=jnp.float32)
        kstart = jnp.clip(q0 - WINDOW, 0, SKV - KW)
        qi = q0 + lax.broadcasted_iota(jnp.int32, (QT, KW), 0)
        ki = kstart + lax.broadcasted_iota(jnp.int32, (QT, KW), 1)
        mask = jnp.abs(qi - ki) <= WINDOW
        krows = pl.ds(pl.multiple_of(kstart, 128), KW)
        for h in range(H_PER):
            scores = lax.dot_general(
                q_t[:, h * DH:(h + 1) * DH].astype(jnp.bfloat16),
                k_ref[h, krows, :], (((1,), (1,)), ((), ())),
                preferred_element_type=jnp.float32) * SCALE
            scores = jnp.where(mask, scores, -1e9)
            m = jnp.max(scores, axis=1, keepdims=True)
            w = jnp.exp(scores - m)
            w = w / jnp.sum(w, axis=1, keepdims=True)
            ctx_ref[rows, h * DH:(h + 1) * DH] = jnp.dot(
                w.astype(jnp.bfloat16), v_ref[h, krows, :],
                preferred_element_type=jnp.float32).astype(jnp.bfloat16)
        p_ref[rows, :] = jnp.dot(ctx_ref[rows, :], wo_ref[...],
                                 preferred_element_type=jnp.float32
                                 ).astype(jnp.bfloat16)
        for j in range(2):
            c = 2 * t + j
            @pl.when(c != my)
            def _():
                pltpu.make_async_remote_copy(
                    src_ref=p_ref.at[pl.ds(c * CHUNK, CHUNK), :],
                    dst_ref=rs_buf.at[my],
                    send_sem=sc_send.at[2 * k + j],
                    recv_sem=sc_recv.at[my],
                    device_id=(c,),
                    device_id_type=pl.DeviceIdType.MESH,
                ).start()

    for p in range(N_DEV):
        @pl.when(p != my)
        def _():
            pltpu.make_async_remote_copy(
                src_ref=rs_buf.at[p], dst_ref=rs_buf.at[p],
                send_sem=sc_send.at[0], recv_sem=sc_recv.at[p],
                device_id=(my,), device_id_type=pl.DeviceIdType.MESH,
            ).wait_recv()
    for k in range(N_TILE):
        t = lax.rem(my_tile + 1 + k, N_TILE)
        for j in range(2):
            c = 2 * t + j
            @pl.when(c != my)
            def _():
                pltpu.make_async_remote_copy(
                    src_ref=p_ref.at[pl.ds(c * CHUNK, CHUNK), :],
                    dst_ref=rs_buf.at[my],
                    send_sem=sc_send.at[2 * k + j],
                    recv_sem=sc_recv.at[my],
                    device_id=(c,), device_id_type=pl.DeviceIdType.MESH,
                ).wait_send()

    own = p_ref[pl.ds(my * CHUNK, CHUNK), :]
    red = jnp.zeros((CHUNK, D_MODEL), jnp.float32)
    for j in range(N_DEV):
        red = red + jnp.where(my == j, own, rs_buf[j]).astype(jnp.float32)
    out_ref[pl.ds(my * CHUNK, CHUNK), :] = red
    bc_src[...] = red.astype(jnp.bfloat16)

    for q in range(N_DEV):
        @pl.when(q != my)
        def _():
            pltpu.make_async_remote_copy(
                src_ref=bc_src,
                dst_ref=bc_buf.at[my],
                send_sem=bc_send.at[q],
                recv_sem=bc_recv.at[my],
                device_id=(q,),
                device_id_type=pl.DeviceIdType.MESH,
            ).start()
    for p in range(N_DEV):
        @pl.when(p != my)
        def _():
            pltpu.make_async_remote_copy(
                src_ref=bc_src, dst_ref=bc_buf.at[p],
                send_sem=bc_send.at[p], recv_sem=bc_recv.at[p],
                device_id=(my,), device_id_type=pl.DeviceIdType.MESH,
            ).wait_recv()
            out_ref[pl.ds(p * CHUNK, CHUNK), :] = (
                bc_buf[p].astype(jnp.float32))
    for q in range(N_DEV):
        @pl.when(q != my)
        def _():
            pltpu.make_async_remote_copy(
                src_ref=bc_src, dst_ref=bc_buf.at[my],
                send_sem=bc_send.at[q], recv_sem=bc_recv.at[my],
                device_id=(q,), device_id_type=pl.DeviceIdType.MESH,
            ).wait_send()


def kernel(x, Wq, K_ext, V_ext, Wo):
    i = lax.axis_index("i")
    x2 = x[0].astype(jnp.bfloat16)
    wq = lax.dynamic_slice(Wq, (0, i * D_MODEL),
                           (D_MODEL, H_PER * DH)).astype(jnp.bfloat16)
    wo = lax.dynamic_slice(Wo, (i * H_PER * DH, 0),
                           (H_PER * DH, D_MODEL)).astype(jnp.bfloat16)
    kh = jnp.swapaxes(K_ext[0], 0, 1).astype(jnp.bfloat16)
    vh = jnp.swapaxes(V_ext[0], 0, 1).astype(jnp.bfloat16)

    out = pl.pallas_call(
        _body,
        out_shape=jax.ShapeDtypeStruct((SQ, D_MODEL), jnp.float32),
        in_specs=[pl.BlockSpec(memory_space=pltpu.VMEM)] * 5,
        out_specs=pl.BlockSpec(memory_space=pltpu.VMEM),
        scratch_shapes=[
            pltpu.VMEM((SQ, H_PER * DH), jnp.bfloat16),
            pltpu.VMEM((SQ, D_MODEL), jnp.bfloat16),
            pltpu.VMEM((CHUNK, D_MODEL), jnp.bfloat16),
            pltpu.VMEM((N_DEV, CHUNK, D_MODEL), jnp.bfloat16),
            pltpu.VMEM((N_DEV, CHUNK, D_MODEL), jnp.bfloat16),
            pltpu.SemaphoreType.DMA((N_DEV,)),
            pltpu.SemaphoreType.DMA((N_DEV,)),
            pltpu.SemaphoreType.DMA((N_DEV,)),
            pltpu.SemaphoreType.DMA((N_DEV,)),
        ],
        compiler_params=pltpu.CompilerParams(collective_id=0),
    )(x2, wq, kh, vh, wo)
    return out[None]
